# baseline (speedup 1.0000x reference)
import functools

import jax
import jax.numpy as jnp
from jax.experimental import pallas as pl
from jax.experimental.pallas import tpu as pltpu

_B, _N, _C, _KNN, _D = 4, 16384, 512, 32, 768
_R = _N // 128
_CB = 8


def _fps_kernel(px_ref, py_ref, pz_ref, cx_ref, cy_ref, cz_ref):
    flat = (jax.lax.broadcasted_iota(jnp.int32, (_R, 128), 0) * 128
            + jax.lax.broadcasted_iota(jnp.int32, (_R, 128), 1))
    px = px_ref[0]
    py = py_ref[0]
    pz = pz_ref[0]
    lx0 = px[0, 0]
    ly0 = py[0, 0]
    lz0 = pz[0, 0]
    cx_ref[pl.ds(0, 1), :] = lx0[None, None]
    cy_ref[pl.ds(0, 1), :] = ly0[None, None]
    cz_ref[pl.ds(0, 1), :] = lz0[None, None]
    dists0 = jnp.full((_R, 128), jnp.inf, jnp.float32)

    def body(i, carry):
        dists, lx, ly, lz = carry
        dx = px - lx
        dy = py - ly
        dz = pz - lz
        d = (dx * dx + dy * dy) + dz * dz
        dists = jnp.minimum(dists, d)
        m = jnp.max(dists)
        sel = jnp.where(dists == m, flat, jnp.int32(1 << 30))
        idx = jnp.min(sel)
        msk = flat == idx
        nlx = jnp.sum(jnp.where(msk, px, 0.0))
        nly = jnp.sum(jnp.where(msk, py, 0.0))
        nlz = jnp.sum(jnp.where(msk, pz, 0.0))
        cx_ref[pl.ds(i, 1), :] = nlx[None, None]
        cy_ref[pl.ds(i, 1), :] = nly[None, None]
        cz_ref[pl.ds(i, 1), :] = nlz[None, None]
        return dists, nlx, nly, nlz

    jax.lax.fori_loop(1, _C, body, (dists0, lx0, ly0, lz0))


@jax.jit
def _fps(px, py, pz):
    cs = pl.pallas_call(
        _fps_kernel,
        grid=(_B,),
        in_specs=[pl.BlockSpec((1, _R, 128), lambda b: (b, 0, 0))] * 3,
        out_specs=[pl.BlockSpec((_C, 1), lambda b: (b, 0))] * 3,
        out_shape=[jax.ShapeDtypeStruct((_B * _C, 1), jnp.float32)] * 3,
    )(px, py, pz)
    return cs


def _knn_kernel(cx_ref, cy_ref, cz_ref, px_ref, py_ref, pz_ref,
                nx_ref, ny_ref, nz_ref, d_scr):
    px = px_ref[0]
    py = py_ref[0]
    pz = pz_ref[0]
    pn = (px * px + py * py) + pz * pz
    pxb = px.astype(jnp.bfloat16).astype(jnp.float32)
    pyb = py.astype(jnp.bfloat16).astype(jnp.float32)
    pzb = pz.astype(jnp.bfloat16).astype(jnp.float32)
    lane = jax.lax.broadcasted_iota(jnp.int32, (1, 128), 1)
    srow = jax.lax.broadcasted_iota(jnp.int32, (_R, 1), 0)
    BIGI = jnp.int32(1 << 30)
    INF = jnp.float32(jnp.inf)
    cxs = [cx_ref[0, 0, i] for i in range(_CB)]
    cys = [cy_ref[0, 0, i] for i in range(_CB)]
    czs = [cz_ref[0, 0, i] for i in range(_CB)]
    Rs = []
    for i in range(_CB):
        cn = (cxs[i] * cxs[i] + cys[i] * cys[i]) + czs[i] * czs[i]
        cxb = cxs[i].astype(jnp.bfloat16).astype(jnp.float32)
        cyb = cys[i].astype(jnp.bfloat16).astype(jnp.float32)
        czb = czs[i].astype(jnp.bfloat16).astype(jnp.float32)
        dot = (cxb * pxb + cyb * pyb) + czb * pzb
        d = cn + pn - 2.0 * dot
        d_scr[i] = d
        Rs.append(jnp.min(d, axis=1, keepdims=True))

    def pass_body(k, Rs):
        Rs = list(Rs)
        for i in range(_CB):
            R = Rs[i]
            m = jnp.min(R)
            r = jnp.min(jnp.where(R == m, srow, BIGI))
            row = d_scr[i, pl.ds(r, 1), :]
            c = jnp.min(jnp.where(row == m, lane, BIGI))
            cm = lane == c
            nxv = jnp.sum(jnp.where(cm, px_ref[0, pl.ds(r, 1), :], 0.0))
            nyv = jnp.sum(jnp.where(cm, py_ref[0, pl.ds(r, 1), :], 0.0))
            nzv = jnp.sum(jnp.where(cm, pz_ref[0, pl.ds(r, 1), :], 0.0))
            nx_ref[pl.ds(i * _KNN + k, 1), :] = (nxv - cxs[i])[None, None]
            ny_ref[pl.ds(i * _KNN + k, 1), :] = (nyv - cys[i])[None, None]
            nz_ref[pl.ds(i * _KNN + k, 1), :] = (nzv - czs[i])[None, None]
            row2 = jnp.where(cm, INF, row)
            d_scr[i, pl.ds(r, 1), :] = row2
            Rs[i] = jnp.where(srow == r, jnp.min(row2), R)
        return tuple(Rs)

    jax.lax.fori_loop(0, _KNN, pass_body, tuple(Rs))


@jax.jit
def _knn_pallas(cx, cy, cz, px, py, pz):
    outs = pl.pallas_call(
        _knn_kernel,
        grid=(_B, _C // _CB),
        in_specs=(
            [pl.BlockSpec((1, 1, _CB), lambda b, c: (b * (_C // _CB) + c, 0, 0),
                          memory_space=pltpu.SMEM)] * 3
            + [pl.BlockSpec((1, _R, 128), lambda b, c: (b, 0, 0))] * 3
        ),
        out_specs=[pl.BlockSpec((_CB * _KNN, 1),
                                lambda b, c: (b * (_C // _CB) + c, 0))] * 3,
        out_shape=[jax.ShapeDtypeStruct((_B * _C * _KNN, 1), jnp.float32)] * 3,
        scratch_shapes=[pltpu.VMEM((_CB, _R, 128), jnp.float32)],
    )(cx, cy, cz, px, py, pz)
    return outs


def _gelu(x):
    return jax.nn.gelu(x, approximate=True)


def _ln(x, g, b):
    m = jnp.mean(x, axis=-1, keepdims=True)
    v = jnp.mean((x - m) ** 2, axis=-1, keepdims=True)
    return (x - m) / jnp.sqrt(v + 1e-5) * g + b


def _knn_jnp(centers, points, k):
    cn = jnp.sum(centers ** 2, axis=-1)[:, :, None]
    pn = jnp.sum(points ** 2, axis=-1)[:, None, :]
    dot = jnp.einsum('bkd,bnd->bkn', centers, points,
                     precision=jax.lax.Precision.HIGHEST)
    d = cn + pn - 2.0 * dot
    _, idx = jax.lax.top_k(-d, k)
    knn_pts = jax.vmap(lambda p, i: jnp.take(p, i, axis=0))(points, idx)
    return knn_pts


def kernel(point_cloud, W1, b1, g1, be1, W2, b2, g2, be2, W3, b3, g3, be3,
           W4, b4, Wc1, bc1, Wc2, bc2):
    px = point_cloud[..., 0].reshape(_B, _R, 128)
    py = point_cloud[..., 1].reshape(_B, _R, 128)
    pz = point_cloud[..., 2].reshape(_B, _R, 128)
    cx, cy, cz = _fps(px, py, pz)
    centers = jnp.concatenate([cx, cy, cz], axis=1).reshape(_B, _C, 3)

    nx, ny, nz = _knn_pallas(cx.reshape(-1, 1, _CB), cy.reshape(-1, 1, _CB),
                             cz.reshape(-1, 1, _CB), px, py, pz)
    normed = jnp.concatenate([nx, ny, nz], axis=1).reshape(_B, _C, _KNN, 3)
    center_emb = _gelu(centers @ Wc1 + bc1) @ Wc2 + bc2
    h = _gelu(_ln(normed @ W1 + b1, g1, be1))
    h = _gelu(_ln(h @ W2 + b2, g2, be2))
    h = _gelu(_ln(h @ W3 + b3, g3, be3))
    h = jnp.max(h, axis=-2)
    knn_emb = h @ W4 + b4
    return (center_emb + knn_emb, centers, normed)


# speedup vs baseline: 4.2136x; 4.2136x over previous
import functools

import jax
import jax.numpy as jnp
from jax.experimental import pallas as pl
from jax.experimental.pallas import tpu as pltpu

_B, _N, _C, _KNN, _D = 4, 16384, 512, 32, 768
_R = _N // 128
_CB = 8


def _fps_kernel(px_ref, py_ref, pz_ref, cx_ref, cy_ref, cz_ref):
    flat = (jax.lax.broadcasted_iota(jnp.int32, (_R, 128), 0) * 128
            + jax.lax.broadcasted_iota(jnp.int32, (_R, 128), 1))
    px = px_ref[0]
    py = py_ref[0]
    pz = pz_ref[0]
    lx0 = px[0, 0]
    ly0 = py[0, 0]
    lz0 = pz[0, 0]
    cx_ref[pl.ds(0, 1), :] = lx0[None, None]
    cy_ref[pl.ds(0, 1), :] = ly0[None, None]
    cz_ref[pl.ds(0, 1), :] = lz0[None, None]
    dists0 = jnp.full((_R, 128), jnp.inf, jnp.float32)

    def body(i, carry):
        dists, lx, ly, lz = carry
        dx = px - lx
        dy = py - ly
        dz = pz - lz
        d = (dx * dx + dy * dy) + dz * dz
        dists = jnp.minimum(dists, d)
        m = jnp.max(dists)
        sel = jnp.where(dists == m, flat, jnp.int32(1 << 30))
        idx = jnp.min(sel)
        msk = flat == idx
        nlx = jnp.sum(jnp.where(msk, px, 0.0))
        nly = jnp.sum(jnp.where(msk, py, 0.0))
        nlz = jnp.sum(jnp.where(msk, pz, 0.0))
        cx_ref[pl.ds(i, 1), :] = nlx[None, None]
        cy_ref[pl.ds(i, 1), :] = nly[None, None]
        cz_ref[pl.ds(i, 1), :] = nlz[None, None]
        return dists, nlx, nly, nlz

    jax.lax.fori_loop(1, _C, body, (dists0, lx0, ly0, lz0))


@jax.jit
def _fps(px, py, pz):
    cs = pl.pallas_call(
        _fps_kernel,
        grid=(_B,),
        in_specs=[pl.BlockSpec((1, _R, 128), lambda b: (b, 0, 0))] * 3,
        out_specs=[pl.BlockSpec((_C, 1), lambda b: (b, 0))] * 3,
        out_shape=[jax.ShapeDtypeStruct((_B * _C, 1), jnp.float32)] * 3,
    )(px, py, pz)
    return cs


def _knn_kernel(cx_ref, cy_ref, cz_ref, px_ref, py_ref, pz_ref,
                pxt_ref, pyt_ref, pzt_ref, pxyz_ref,
                nx_ref, ny_ref, nz_ref, d_scr):
    px = px_ref[0]
    py = py_ref[0]
    pz = pz_ref[0]
    pn = (px * px + py * py) + pz * pz
    pxt = pxt_ref[0]
    pyt = pyt_ref[0]
    pzt = pzt_ref[0]
    pnt = (pxt * pxt + pyt * pyt) + pzt * pzt
    pxb = px.astype(jnp.bfloat16).astype(jnp.float32)
    pyb = py.astype(jnp.bfloat16).astype(jnp.float32)
    pzb = pz.astype(jnp.bfloat16).astype(jnp.float32)
    pxtb = pxt.astype(jnp.bfloat16).astype(jnp.float32)
    pytb = pyt.astype(jnp.bfloat16).astype(jnp.float32)
    pztb = pzt.astype(jnp.bfloat16).astype(jnp.float32)
    lane = jax.lax.broadcasted_iota(jnp.int32, (1, 128), 1)
    lane3 = jax.lax.broadcasted_iota(jnp.int32, (1, 3 * 128), 1)
    lane32 = jax.lax.broadcasted_iota(jnp.int32, (1, _KNN), 1)
    BIGI = jnp.int32(1 << 30)
    INF = jnp.float32(jnp.inf)
    cxs = [cx_ref[0, 0, i] for i in range(_CB)]
    cys = [cy_ref[0, 0, i] for i in range(_CB)]
    czs = [cz_ref[0, 0, i] for i in range(_CB)]
    rows_R = []
    for i in range(_CB):
        cn = (cxs[i] * cxs[i] + cys[i] * cys[i]) + czs[i] * czs[i]
        cxb = cxs[i].astype(jnp.bfloat16).astype(jnp.float32)
        cyb = cys[i].astype(jnp.bfloat16).astype(jnp.float32)
        czb = czs[i].astype(jnp.bfloat16).astype(jnp.float32)
        dot = (cxb * pxb + cyb * pyb) + czb * pzb
        d_scr[i] = cn + pn - 2.0 * dot
        dott = (cxb * pxtb + cyb * pytb) + czb * pztb
        dt = cn + pnt - 2.0 * dott
        rows_R.append(jnp.min(dt, axis=0, keepdims=True))
    R8 = jnp.concatenate(rows_R, axis=0)
    cx8 = jnp.concatenate([c[None, None] for c in cxs], axis=0)
    cy8 = jnp.concatenate([c[None, None] for c in cys], axis=0)
    cz8 = jnp.concatenate([c[None, None] for c in czs], axis=0)

    def pass_body(k, carry):
        R8, ax, ay, az = carry
        m8 = jnp.min(R8, axis=1, keepdims=True)
        g8 = jnp.min(jnp.where(R8 == m8, lane, BIGI), axis=1,
                     keepdims=True)
        gs = [g8[i, 0] for i in range(_CB)]
        rows = jnp.concatenate(
            [d_scr[i, pl.ds(gs[i], 1), :] for i in range(_CB)], axis=0)
        c8 = jnp.min(jnp.where(rows == m8, lane, BIGI), axis=1,
                     keepdims=True)
        prows = jnp.concatenate(
            [pxyz_ref[0, pl.ds(gs[i], 1), :] for i in range(_CB)], axis=0)
        nx8 = jnp.sum(jnp.where(lane3 == c8, prows, 0.0), axis=1,
                      keepdims=True)
        ny8 = jnp.sum(jnp.where(lane3 == c8 + 128, prows, 0.0), axis=1,
                      keepdims=True)
        nz8 = jnp.sum(jnp.where(lane3 == c8 + 256, prows, 0.0), axis=1,
                      keepdims=True)
        km = lane32 == k
        ax = jnp.where(km, nx8 - cx8, ax)
        ay = jnp.where(km, ny8 - cy8, ay)
        az = jnp.where(km, nz8 - cz8, az)
        rows2 = jnp.where(lane == c8, INF, rows)
        for i in range(_CB):
            d_scr[i, pl.ds(gs[i], 1), :] = rows2[i:i + 1, :]
        R8 = jnp.where(lane == g8, jnp.min(rows2, axis=1, keepdims=True), R8)
        return R8, ax, ay, az

    z32 = jnp.zeros((_CB, _KNN), jnp.float32)
    _, ax, ay, az = jax.lax.fori_loop(0, _KNN, pass_body, (R8, z32, z32, z32))
    nx_ref[...] = ax
    ny_ref[...] = ay
    nz_ref[...] = az


@jax.jit
def _knn_pallas(cx, cy, cz, px, py, pz, pxt, pyt, pzt, pxyz):
    outs = pl.pallas_call(
        _knn_kernel,
        grid=(_B, _C // _CB),
        in_specs=(
            [pl.BlockSpec((1, 1, _CB), lambda b, c: (b * (_C // _CB) + c, 0, 0),
                          memory_space=pltpu.SMEM)] * 3
            + [pl.BlockSpec((1, _R, 128), lambda b, c: (b, 0, 0))] * 3
            + [pl.BlockSpec((1, 128, _R), lambda b, c: (b, 0, 0))] * 3
            + [pl.BlockSpec((1, _R, 3 * 128), lambda b, c: (b, 0, 0))]
        ),
        out_specs=[pl.BlockSpec((_CB, _KNN),
                                lambda b, c: (b * (_C // _CB) + c, 0))] * 3,
        out_shape=[jax.ShapeDtypeStruct((_B * _C, _KNN), jnp.float32)] * 3,
        scratch_shapes=[pltpu.VMEM((_CB, _R, 128), jnp.float32)],
    )(cx, cy, cz, px, py, pz, pxt, pyt, pzt, pxyz)
    return outs


def _gelu(x):
    return jax.nn.gelu(x, approximate=True)


def _ln(x, g, b):
    m = jnp.mean(x, axis=-1, keepdims=True)
    v = jnp.mean((x - m) ** 2, axis=-1, keepdims=True)
    return (x - m) / jnp.sqrt(v + 1e-5) * g + b


def _knn_jnp(centers, points, k):
    cn = jnp.sum(centers ** 2, axis=-1)[:, :, None]
    pn = jnp.sum(points ** 2, axis=-1)[:, None, :]
    dot = jnp.einsum('bkd,bnd->bkn', centers, points,
                     precision=jax.lax.Precision.HIGHEST)
    d = cn + pn - 2.0 * dot
    _, idx = jax.lax.top_k(-d, k)
    knn_pts = jax.vmap(lambda p, i: jnp.take(p, i, axis=0))(points, idx)
    return knn_pts


def kernel(point_cloud, W1, b1, g1, be1, W2, b2, g2, be2, W3, b3, g3, be3,
           W4, b4, Wc1, bc1, Wc2, bc2):
    px = point_cloud[..., 0].reshape(_B, _R, 128)
    py = point_cloud[..., 1].reshape(_B, _R, 128)
    pz = point_cloud[..., 2].reshape(_B, _R, 128)
    cx, cy, cz = _fps(px, py, pz)
    centers = jnp.concatenate([cx, cy, cz], axis=1).reshape(_B, _C, 3)

    pxt = jnp.swapaxes(px, 1, 2)
    pyt = jnp.swapaxes(py, 1, 2)
    pzt = jnp.swapaxes(pz, 1, 2)
    pxyz = jnp.concatenate([px, py, pz], axis=2)
    nx, ny, nz = _knn_pallas(cx.reshape(-1, 1, _CB), cy.reshape(-1, 1, _CB),
                             cz.reshape(-1, 1, _CB), px, py, pz,
                             pxt, pyt, pzt, pxyz)
    normed = jnp.stack([nx, ny, nz], axis=-1).reshape(_B, _C, _KNN, 3)
    center_emb = _gelu(centers @ Wc1 + bc1) @ Wc2 + bc2
    h = _gelu(_ln(normed @ W1 + b1, g1, be1))
    h = _gelu(_ln(h @ W2 + b2, g2, be2))
    h = _gelu(_ln(h @ W3 + b3, g3, be3))
    h = jnp.max(h, axis=-2)
    knn_emb = h @ W4 + b4
    return (center_emb + knn_emb, centers, normed)


# speedup vs baseline: 4.2192x; 1.0013x over previous
import functools

import jax
import jax.numpy as jnp
from jax.experimental import pallas as pl
from jax.experimental.pallas import tpu as pltpu

_B, _N, _C, _KNN, _D = 4, 16384, 512, 32, 768
_R = _N // 128
_CB = 8


def _fps_kernel(px_ref, py_ref, pz_ref, cx_ref, cy_ref, cz_ref):
    flat = (jax.lax.broadcasted_iota(jnp.int32, (_R, 128), 0) * 128
            + jax.lax.broadcasted_iota(jnp.int32, (_R, 128), 1))
    px = px_ref[0]
    py = py_ref[0]
    pz = pz_ref[0]
    lx0 = px[0, 0]
    ly0 = py[0, 0]
    lz0 = pz[0, 0]
    cx_ref[pl.ds(0, 1), :] = lx0[None, None]
    cy_ref[pl.ds(0, 1), :] = ly0[None, None]
    cz_ref[pl.ds(0, 1), :] = lz0[None, None]
    dists0 = jnp.full((_R, 128), jnp.inf, jnp.float32)

    def body(i, carry):
        dists, lx, ly, lz = carry
        dx = px - lx
        dy = py - ly
        dz = pz - lz
        d = (dx * dx + dy * dy) + dz * dz
        dists = jnp.minimum(dists, d)
        m = jnp.max(dists)
        sel = jnp.where(dists == m, flat, jnp.int32(1 << 30))
        idx = jnp.min(sel)
        msk = flat == idx
        nlx = jnp.sum(jnp.where(msk, px, 0.0))
        nly = jnp.sum(jnp.where(msk, py, 0.0))
        nlz = jnp.sum(jnp.where(msk, pz, 0.0))
        cx_ref[pl.ds(i, 1), :] = nlx[None, None]
        cy_ref[pl.ds(i, 1), :] = nly[None, None]
        cz_ref[pl.ds(i, 1), :] = nlz[None, None]
        return dists, nlx, nly, nlz

    jax.lax.fori_loop(1, _C, body, (dists0, lx0, ly0, lz0))


@jax.jit
def _fps(px, py, pz):
    cs = pl.pallas_call(
        _fps_kernel,
        grid=(_B,),
        in_specs=[pl.BlockSpec((1, _R, 128), lambda b: (b, 0, 0))] * 3,
        out_specs=[pl.BlockSpec((_C, 1), lambda b: (b, 0))] * 3,
        out_shape=[jax.ShapeDtypeStruct((_B * _C, 1), jnp.float32)] * 3,
        compiler_params=pltpu.CompilerParams(
            dimension_semantics=("parallel",)),
    )(px, py, pz)
    return cs


def _knn_kernel(cx_ref, cy_ref, cz_ref, px_ref, py_ref, pz_ref,
                pxt_ref, pyt_ref, pzt_ref, pxyz_ref,
                nx_ref, ny_ref, nz_ref, d_scr):
    px = px_ref[0]
    py = py_ref[0]
    pz = pz_ref[0]
    pn = (px * px + py * py) + pz * pz
    pxt = pxt_ref[0]
    pyt = pyt_ref[0]
    pzt = pzt_ref[0]
    pnt = (pxt * pxt + pyt * pyt) + pzt * pzt
    pxb = px.astype(jnp.bfloat16).astype(jnp.float32)
    pyb = py.astype(jnp.bfloat16).astype(jnp.float32)
    pzb = pz.astype(jnp.bfloat16).astype(jnp.float32)
    pxtb = pxt.astype(jnp.bfloat16).astype(jnp.float32)
    pytb = pyt.astype(jnp.bfloat16).astype(jnp.float32)
    pztb = pzt.astype(jnp.bfloat16).astype(jnp.float32)
    lane = jax.lax.broadcasted_iota(jnp.int32, (1, 128), 1)
    lane3 = jax.lax.broadcasted_iota(jnp.int32, (1, 3 * 128), 1)
    lane32 = jax.lax.broadcasted_iota(jnp.int32, (1, _KNN), 1)
    BIGI = jnp.int32(1 << 30)
    INF = jnp.float32(jnp.inf)
    cxs = [cx_ref[0, 0, i] for i in range(_CB)]
    cys = [cy_ref[0, 0, i] for i in range(_CB)]
    czs = [cz_ref[0, 0, i] for i in range(_CB)]
    rows_R = []
    for i in range(_CB):
        cn = (cxs[i] * cxs[i] + cys[i] * cys[i]) + czs[i] * czs[i]
        cxb = cxs[i].astype(jnp.bfloat16).astype(jnp.float32)
        cyb = cys[i].astype(jnp.bfloat16).astype(jnp.float32)
        czb = czs[i].astype(jnp.bfloat16).astype(jnp.float32)
        dot = (cxb * pxb + cyb * pyb) + czb * pzb
        d_scr[i] = cn + pn - 2.0 * dot
        dott = (cxb * pxtb + cyb * pytb) + czb * pztb
        dt = cn + pnt - 2.0 * dott
        rows_R.append(jnp.min(dt, axis=0, keepdims=True))
    R8 = jnp.concatenate(rows_R, axis=0)
    cx8 = jnp.concatenate([c[None, None] for c in cxs], axis=0)
    cy8 = jnp.concatenate([c[None, None] for c in cys], axis=0)
    cz8 = jnp.concatenate([c[None, None] for c in czs], axis=0)

    def pass_body(k, carry):
        R8, ax, ay, az = carry
        m8 = jnp.min(R8, axis=1, keepdims=True)
        g8 = jnp.min(jnp.where(R8 == m8, lane, BIGI), axis=1,
                     keepdims=True)
        gs = [g8[i, 0] for i in range(_CB)]
        rows = jnp.concatenate(
            [d_scr[i, pl.ds(gs[i], 1), :] for i in range(_CB)], axis=0)
        c8 = jnp.min(jnp.where(rows == m8, lane, BIGI), axis=1,
                     keepdims=True)
        prows = jnp.concatenate(
            [pxyz_ref[0, pl.ds(gs[i], 1), :] for i in range(_CB)], axis=0)
        nx8 = jnp.sum(jnp.where(lane3 == c8, prows, 0.0), axis=1,
                      keepdims=True)
        ny8 = jnp.sum(jnp.where(lane3 == c8 + 128, prows, 0.0), axis=1,
                      keepdims=True)
        nz8 = jnp.sum(jnp.where(lane3 == c8 + 256, prows, 0.0), axis=1,
                      keepdims=True)
        km = lane32 == k
        ax = jnp.where(km, nx8 - cx8, ax)
        ay = jnp.where(km, ny8 - cy8, ay)
        az = jnp.where(km, nz8 - cz8, az)
        rows2 = jnp.where(lane == c8, INF, rows)
        for i in range(_CB):
            d_scr[i, pl.ds(gs[i], 1), :] = rows2[i:i + 1, :]
        R8 = jnp.where(lane == g8, jnp.min(rows2, axis=1, keepdims=True), R8)
        return R8, ax, ay, az

    z32 = jnp.zeros((_CB, _KNN), jnp.float32)
    _, ax, ay, az = jax.lax.fori_loop(0, _KNN, pass_body, (R8, z32, z32, z32))
    nx_ref[...] = ax
    ny_ref[...] = ay
    nz_ref[...] = az


@jax.jit
def _knn_pallas(cx, cy, cz, px, py, pz, pxt, pyt, pzt, pxyz):
    outs = pl.pallas_call(
        _knn_kernel,
        grid=(_B, _C // _CB),
        in_specs=(
            [pl.BlockSpec((1, 1, _CB), lambda b, c: (b * (_C // _CB) + c, 0, 0),
                          memory_space=pltpu.SMEM)] * 3
            + [pl.BlockSpec((1, _R, 128), lambda b, c: (b, 0, 0))] * 3
            + [pl.BlockSpec((1, 128, _R), lambda b, c: (b, 0, 0))] * 3
            + [pl.BlockSpec((1, _R, 3 * 128), lambda b, c: (b, 0, 0))]
        ),
        out_specs=[pl.BlockSpec((_CB, _KNN),
                                lambda b, c: (b * (_C // _CB) + c, 0))] * 3,
        out_shape=[jax.ShapeDtypeStruct((_B * _C, _KNN), jnp.float32)] * 3,
        scratch_shapes=[pltpu.VMEM((_CB, _R, 128), jnp.float32)],
        compiler_params=pltpu.CompilerParams(
            dimension_semantics=("parallel", "parallel")),
    )(cx, cy, cz, px, py, pz, pxt, pyt, pzt, pxyz)
    return outs


def _gelu(x):
    return jax.nn.gelu(x, approximate=True)


def _ln(x, g, b):
    m = jnp.mean(x, axis=-1, keepdims=True)
    v = jnp.mean((x - m) ** 2, axis=-1, keepdims=True)
    return (x - m) / jnp.sqrt(v + 1e-5) * g + b


def _knn_jnp(centers, points, k):
    cn = jnp.sum(centers ** 2, axis=-1)[:, :, None]
    pn = jnp.sum(points ** 2, axis=-1)[:, None, :]
    dot = jnp.einsum('bkd,bnd->bkn', centers, points,
                     precision=jax.lax.Precision.HIGHEST)
    d = cn + pn - 2.0 * dot
    _, idx = jax.lax.top_k(-d, k)
    knn_pts = jax.vmap(lambda p, i: jnp.take(p, i, axis=0))(points, idx)
    return knn_pts


def kernel(point_cloud, W1, b1, g1, be1, W2, b2, g2, be2, W3, b3, g3, be3,
           W4, b4, Wc1, bc1, Wc2, bc2):
    px = point_cloud[..., 0].reshape(_B, _R, 128)
    py = point_cloud[..., 1].reshape(_B, _R, 128)
    pz = point_cloud[..., 2].reshape(_B, _R, 128)
    cx, cy, cz = _fps(px, py, pz)
    centers = jnp.concatenate([cx, cy, cz], axis=1).reshape(_B, _C, 3)

    pxt = jnp.swapaxes(px, 1, 2)
    pyt = jnp.swapaxes(py, 1, 2)
    pzt = jnp.swapaxes(pz, 1, 2)
    pxyz = jnp.concatenate([px, py, pz], axis=2)
    nx, ny, nz = _knn_pallas(cx.reshape(-1, 1, _CB), cy.reshape(-1, 1, _CB),
                             cz.reshape(-1, 1, _CB), px, py, pz,
                             pxt, pyt, pzt, pxyz)
    normed = jnp.stack([nx, ny, nz], axis=-1).reshape(_B, _C, _KNN, 3)
    center_emb = _gelu(centers @ Wc1 + bc1) @ Wc2 + bc2
    h = _gelu(_ln(normed @ W1 + b1, g1, be1))
    h = _gelu(_ln(h @ W2 + b2, g2, be2))
    h = _gelu(_ln(h @ W3 + b3, g3, be3))
    h = jnp.max(h, axis=-2)
    knn_emb = h @ W4 + b4
    return (center_emb + knn_emb, centers, normed)


# speedup vs baseline: 6.8002x; 1.6117x over previous
import functools

import jax
import jax.numpy as jnp
from jax.experimental import pallas as pl
from jax.experimental.pallas import tpu as pltpu

_B, _N, _C, _KNN, _D = 4, 16384, 512, 32, 768
_R = _N // 128
_CB = 16


def _fps_kernel(px_ref, py_ref, pz_ref, cx_ref, cy_ref, cz_ref):
    flat = (jax.lax.broadcasted_iota(jnp.int32, (_R, 128), 0) * 128
            + jax.lax.broadcasted_iota(jnp.int32, (_R, 128), 1))
    px = px_ref[0]
    py = py_ref[0]
    pz = pz_ref[0]
    lx0 = px[0, 0]
    ly0 = py[0, 0]
    lz0 = pz[0, 0]
    cx_ref[pl.ds(0, 1), :] = lx0[None, None]
    cy_ref[pl.ds(0, 1), :] = ly0[None, None]
    cz_ref[pl.ds(0, 1), :] = lz0[None, None]
    dists0 = jnp.full((_R, 128), jnp.inf, jnp.float32)

    def body(i, carry):
        dists, lx, ly, lz = carry
        dx = px - lx
        dy = py - ly
        dz = pz - lz
        d = (dx * dx + dy * dy) + dz * dz
        dists = jnp.minimum(dists, d)
        m = jnp.max(dists)
        sel = jnp.where(dists == m, flat, jnp.int32(1 << 30))
        idx = jnp.min(sel)
        msk = flat == idx
        nlx = jnp.sum(jnp.where(msk, px, 0.0))
        nly = jnp.sum(jnp.where(msk, py, 0.0))
        nlz = jnp.sum(jnp.where(msk, pz, 0.0))
        cx_ref[pl.ds(i, 1), :] = nlx[None, None]
        cy_ref[pl.ds(i, 1), :] = nly[None, None]
        cz_ref[pl.ds(i, 1), :] = nlz[None, None]
        return dists, nlx, nly, nlz

    jax.lax.fori_loop(1, _C, body, (dists0, lx0, ly0, lz0))


@jax.jit
def _fps(px, py, pz):
    cs = pl.pallas_call(
        _fps_kernel,
        grid=(_B,),
        in_specs=[pl.BlockSpec((1, _R, 128), lambda b: (b, 0, 0))] * 3,
        out_specs=[pl.BlockSpec((_C, 1), lambda b: (b, 0))] * 3,
        out_shape=[jax.ShapeDtypeStruct((_B * _C, 1), jnp.float32)] * 3,
        compiler_params=pltpu.CompilerParams(
            dimension_semantics=("parallel",)),
    )(px, py, pz)
    return cs


def _knn_kernel(cx_ref, cy_ref, cz_ref, px_ref, py_ref, pz_ref,
                pxt_ref, pyt_ref, pzt_ref, pxyz_ref,
                nx_ref, ny_ref, nz_ref, d_scr):
    px = px_ref[0]
    py = py_ref[0]
    pz = pz_ref[0]
    pn = (px * px + py * py) + pz * pz
    pxt = pxt_ref[0]
    pyt = pyt_ref[0]
    pzt = pzt_ref[0]
    pnt = (pxt * pxt + pyt * pyt) + pzt * pzt
    pxb = px.astype(jnp.bfloat16).astype(jnp.float32)
    pyb = py.astype(jnp.bfloat16).astype(jnp.float32)
    pzb = pz.astype(jnp.bfloat16).astype(jnp.float32)
    pxtb = pxt.astype(jnp.bfloat16).astype(jnp.float32)
    pytb = pyt.astype(jnp.bfloat16).astype(jnp.float32)
    pztb = pzt.astype(jnp.bfloat16).astype(jnp.float32)
    lane = jax.lax.broadcasted_iota(jnp.int32, (1, 128), 1)
    lane3 = jax.lax.broadcasted_iota(jnp.int32, (1, 3 * 128), 1)
    lane32 = jax.lax.broadcasted_iota(jnp.int32, (1, _KNN), 1)
    BIGI = jnp.int32(1 << 30)
    INF = jnp.float32(jnp.inf)
    cxs = [cx_ref[0, 0, i] for i in range(_CB)]
    cys = [cy_ref[0, 0, i] for i in range(_CB)]
    czs = [cz_ref[0, 0, i] for i in range(_CB)]
    rows_R = []
    for i in range(_CB):
        cn = (cxs[i] * cxs[i] + cys[i] * cys[i]) + czs[i] * czs[i]
        cxb = cxs[i].astype(jnp.bfloat16).astype(jnp.float32)
        cyb = cys[i].astype(jnp.bfloat16).astype(jnp.float32)
        czb = czs[i].astype(jnp.bfloat16).astype(jnp.float32)
        dot = (cxb * pxb + cyb * pyb) + czb * pzb
        d_scr[i] = cn + pn - 2.0 * dot
        dott = (cxb * pxtb + cyb * pytb) + czb * pztb
        dt = cn + pnt - 2.0 * dott
        rows_R.append(jnp.min(dt, axis=0, keepdims=True))
    R8 = jnp.concatenate(rows_R, axis=0)
    cx8 = jnp.concatenate([c[None, None] for c in cxs], axis=0)
    cy8 = jnp.concatenate([c[None, None] for c in cys], axis=0)
    cz8 = jnp.concatenate([c[None, None] for c in czs], axis=0)

    def pass_body(k, carry):
        R8, ax, ay, az = carry
        m8 = jnp.min(R8, axis=1, keepdims=True)
        g8 = jnp.min(jnp.where(R8 == m8, lane, BIGI), axis=1,
                     keepdims=True)
        gs = [g8[i, 0] for i in range(_CB)]
        rows = jnp.concatenate(
            [d_scr[i, pl.ds(gs[i], 1), :] for i in range(_CB)], axis=0)
        c8 = jnp.min(jnp.where(rows == m8, lane, BIGI), axis=1,
                     keepdims=True)
        prows = jnp.concatenate(
            [pxyz_ref[0, pl.ds(gs[i], 1), :] for i in range(_CB)], axis=0)
        nx8 = jnp.sum(jnp.where(lane3 == c8, prows, 0.0), axis=1,
                      keepdims=True)
        ny8 = jnp.sum(jnp.where(lane3 == c8 + 128, prows, 0.0), axis=1,
                      keepdims=True)
        nz8 = jnp.sum(jnp.where(lane3 == c8 + 256, prows, 0.0), axis=1,
                      keepdims=True)
        km = lane32 == k
        ax = jnp.where(km, nx8 - cx8, ax)
        ay = jnp.where(km, ny8 - cy8, ay)
        az = jnp.where(km, nz8 - cz8, az)
        rows2 = jnp.where(lane == c8, INF, rows)
        for i in range(_CB):
            d_scr[i, pl.ds(gs[i], 1), :] = rows2[i:i + 1, :]
        R8 = jnp.where(lane == g8, jnp.min(rows2, axis=1, keepdims=True), R8)
        return R8, ax, ay, az

    z32 = jnp.zeros((_CB, _KNN), jnp.float32)
    _, ax, ay, az = jax.lax.fori_loop(0, _KNN, pass_body, (R8, z32, z32, z32))
    nx_ref[...] = ax
    ny_ref[...] = ay
    nz_ref[...] = az


@jax.jit
def _knn_pallas(cx, cy, cz, px, py, pz, pxt, pyt, pzt, pxyz):
    outs = pl.pallas_call(
        _knn_kernel,
        grid=(_B, _C // _CB),
        in_specs=(
            [pl.BlockSpec((1, 1, _CB), lambda b, c: (b * (_C // _CB) + c, 0, 0),
                          memory_space=pltpu.SMEM)] * 3
            + [pl.BlockSpec((1, _R, 128), lambda b, c: (b, 0, 0))] * 3
            + [pl.BlockSpec((1, 128, _R), lambda b, c: (b, 0, 0))] * 3
            + [pl.BlockSpec((1, _R, 3 * 128), lambda b, c: (b, 0, 0))]
        ),
        out_specs=[pl.BlockSpec((_CB, _KNN),
                                lambda b, c: (b * (_C // _CB) + c, 0))] * 3,
        out_shape=[jax.ShapeDtypeStruct((_B * _C, _KNN), jnp.float32)] * 3,
        scratch_shapes=[pltpu.VMEM((_CB, _R, 128), jnp.float32)],
        compiler_params=pltpu.CompilerParams(
            dimension_semantics=("parallel", "parallel")),
    )(cx, cy, cz, px, py, pz, pxt, pyt, pzt, pxyz)
    return outs


def _gelu(x):
    return jax.nn.gelu(x, approximate=True)


def _ln(x, g, b):
    m = jnp.mean(x, axis=-1, keepdims=True)
    v = jnp.mean((x - m) ** 2, axis=-1, keepdims=True)
    return (x - m) / jnp.sqrt(v + 1e-5) * g + b


def _knn_jnp(centers, points, k):
    cn = jnp.sum(centers ** 2, axis=-1)[:, :, None]
    pn = jnp.sum(points ** 2, axis=-1)[:, None, :]
    dot = jnp.einsum('bkd,bnd->bkn', centers, points,
                     precision=jax.lax.Precision.HIGHEST)
    d = cn + pn - 2.0 * dot
    _, idx = jax.lax.top_k(-d, k)
    knn_pts = jax.vmap(lambda p, i: jnp.take(p, i, axis=0))(points, idx)
    return knn_pts


def kernel(point_cloud, W1, b1, g1, be1, W2, b2, g2, be2, W3, b3, g3, be3,
           W4, b4, Wc1, bc1, Wc2, bc2):
    px = point_cloud[..., 0].reshape(_B, _R, 128)
    py = point_cloud[..., 1].reshape(_B, _R, 128)
    pz = point_cloud[..., 2].reshape(_B, _R, 128)
    cx, cy, cz = _fps(px, py, pz)
    centers = jnp.concatenate([cx, cy, cz], axis=1).reshape(_B, _C, 3)

    pxt = jnp.swapaxes(px, 1, 2)
    pyt = jnp.swapaxes(py, 1, 2)
    pzt = jnp.swapaxes(pz, 1, 2)
    pxyz = jnp.concatenate([px, py, pz], axis=2)
    nx, ny, nz = _knn_pallas(cx.reshape(-1, 1, _CB), cy.reshape(-1, 1, _CB),
                             cz.reshape(-1, 1, _CB), px, py, pz,
                             pxt, pyt, pzt, pxyz)
    normed = jnp.stack([nx, ny, nz], axis=-1).reshape(_B, _C, _KNN, 3)
    center_emb = _gelu(centers @ Wc1 + bc1) @ Wc2 + bc2
    h = _gelu(_ln(normed @ W1 + b1, g1, be1))
    h = _gelu(_ln(h @ W2 + b2, g2, be2))
    h = _gelu(_ln(h @ W3 + b3, g3, be3))
    h = jnp.max(h, axis=-2)
    knn_emb = h @ W4 + b4
    return (center_emb + knn_emb, centers, normed)


# speedup vs baseline: 10.0578x; 1.4790x over previous
import functools

import jax
import jax.numpy as jnp
from jax.experimental import pallas as pl
from jax.experimental.pallas import tpu as pltpu

_B, _N, _C, _KNN, _D = 4, 16384, 512, 32, 768
_R = _N // 128
_CB = 32


def _fps_kernel(px_ref, py_ref, pz_ref, cx_ref, cy_ref, cz_ref):
    flat = (jax.lax.broadcasted_iota(jnp.int32, (_R, 128), 0) * 128
            + jax.lax.broadcasted_iota(jnp.int32, (_R, 128), 1))
    px = px_ref[0]
    py = py_ref[0]
    pz = pz_ref[0]
    lx0 = px[0, 0]
    ly0 = py[0, 0]
    lz0 = pz[0, 0]
    cx_ref[pl.ds(0, 1), :] = lx0[None, None]
    cy_ref[pl.ds(0, 1), :] = ly0[None, None]
    cz_ref[pl.ds(0, 1), :] = lz0[None, None]
    dists0 = jnp.full((_R, 128), jnp.inf, jnp.float32)

    def body(i, carry):
        dists, lx, ly, lz = carry
        dx = px - lx
        dy = py - ly
        dz = pz - lz
        d = (dx * dx + dy * dy) + dz * dz
        dists = jnp.minimum(dists, d)
        m = jnp.max(dists)
        sel = jnp.where(dists == m, flat, jnp.int32(1 << 30))
        idx = jnp.min(sel)
        msk = flat == idx
        nlx = jnp.sum(jnp.where(msk, px, 0.0))
        nly = jnp.sum(jnp.where(msk, py, 0.0))
        nlz = jnp.sum(jnp.where(msk, pz, 0.0))
        cx_ref[pl.ds(i, 1), :] = nlx[None, None]
        cy_ref[pl.ds(i, 1), :] = nly[None, None]
        cz_ref[pl.ds(i, 1), :] = nlz[None, None]
        return dists, nlx, nly, nlz

    jax.lax.fori_loop(1, _C, body, (dists0, lx0, ly0, lz0))


@jax.jit
def _fps(px, py, pz):
    cs = pl.pallas_call(
        _fps_kernel,
        grid=(_B,),
        in_specs=[pl.BlockSpec((1, _R, 128), lambda b: (b, 0, 0))] * 3,
        out_specs=[pl.BlockSpec((_C, 1), lambda b: (b, 0))] * 3,
        out_shape=[jax.ShapeDtypeStruct((_B * _C, 1), jnp.float32)] * 3,
        compiler_params=pltpu.CompilerParams(
            dimension_semantics=("parallel",)),
    )(px, py, pz)
    return cs


def _knn_kernel(cx_ref, cy_ref, cz_ref, px_ref, py_ref, pz_ref,
                pxt_ref, pyt_ref, pzt_ref, pxyz_ref,
                nx_ref, ny_ref, nz_ref, d_scr):
    px = px_ref[0]
    py = py_ref[0]
    pz = pz_ref[0]
    pn = (px * px + py * py) + pz * pz
    pxt = pxt_ref[0]
    pyt = pyt_ref[0]
    pzt = pzt_ref[0]
    pnt = (pxt * pxt + pyt * pyt) + pzt * pzt
    pxb = px.astype(jnp.bfloat16).astype(jnp.float32)
    pyb = py.astype(jnp.bfloat16).astype(jnp.float32)
    pzb = pz.astype(jnp.bfloat16).astype(jnp.float32)
    pxtb = pxt.astype(jnp.bfloat16).astype(jnp.float32)
    pytb = pyt.astype(jnp.bfloat16).astype(jnp.float32)
    pztb = pzt.astype(jnp.bfloat16).astype(jnp.float32)
    lane = jax.lax.broadcasted_iota(jnp.int32, (1, 128), 1)
    lane3 = jax.lax.broadcasted_iota(jnp.int32, (1, 3 * 128), 1)
    lane32 = jax.lax.broadcasted_iota(jnp.int32, (1, _KNN), 1)
    BIGI = jnp.int32(1 << 30)
    INF = jnp.float32(jnp.inf)
    cxs = [cx_ref[0, 0, i] for i in range(_CB)]
    cys = [cy_ref[0, 0, i] for i in range(_CB)]
    czs = [cz_ref[0, 0, i] for i in range(_CB)]
    rows_R = []
    for i in range(_CB):
        cn = (cxs[i] * cxs[i] + cys[i] * cys[i]) + czs[i] * czs[i]
        cxb = cxs[i].astype(jnp.bfloat16).astype(jnp.float32)
        cyb = cys[i].astype(jnp.bfloat16).astype(jnp.float32)
        czb = czs[i].astype(jnp.bfloat16).astype(jnp.float32)
        dot = (cxb * pxb + cyb * pyb) + czb * pzb
        d_scr[i] = cn + pn - 2.0 * dot
        dott = (cxb * pxtb + cyb * pytb) + czb * pztb
        dt = cn + pnt - 2.0 * dott
        rows_R.append(jnp.min(dt, axis=0, keepdims=True))
    R8 = jnp.concatenate(rows_R, axis=0)
    cx8 = jnp.concatenate([c[None, None] for c in cxs], axis=0)
    cy8 = jnp.concatenate([c[None, None] for c in cys], axis=0)
    cz8 = jnp.concatenate([c[None, None] for c in czs], axis=0)

    def pass_body(k, carry):
        R8, ax, ay, az = carry
        m8 = jnp.min(R8, axis=1, keepdims=True)
        g8 = jnp.min(jnp.where(R8 == m8, lane, BIGI), axis=1,
                     keepdims=True)
        gs = [g8[i, 0] for i in range(_CB)]
        rows = jnp.concatenate(
            [d_scr[i, pl.ds(gs[i], 1), :] for i in range(_CB)], axis=0)
        c8 = jnp.min(jnp.where(rows == m8, lane, BIGI), axis=1,
                     keepdims=True)
        prows = jnp.concatenate(
            [pxyz_ref[0, pl.ds(gs[i], 1), :] for i in range(_CB)], axis=0)
        nx8 = jnp.sum(jnp.where(lane3 == c8, prows, 0.0), axis=1,
                      keepdims=True)
        ny8 = jnp.sum(jnp.where(lane3 == c8 + 128, prows, 0.0), axis=1,
                      keepdims=True)
        nz8 = jnp.sum(jnp.where(lane3 == c8 + 256, prows, 0.0), axis=1,
                      keepdims=True)
        km = lane32 == k
        ax = jnp.where(km, nx8 - cx8, ax)
        ay = jnp.where(km, ny8 - cy8, ay)
        az = jnp.where(km, nz8 - cz8, az)
        rows2 = jnp.where(lane == c8, INF, rows)
        for i in range(_CB):
            d_scr[i, pl.ds(gs[i], 1), :] = rows2[i:i + 1, :]
        R8 = jnp.where(lane == g8, jnp.min(rows2, axis=1, keepdims=True), R8)
        return R8, ax, ay, az

    z32 = jnp.zeros((_CB, _KNN), jnp.float32)
    _, ax, ay, az = jax.lax.fori_loop(0, _KNN, pass_body, (R8, z32, z32, z32))
    nx_ref[...] = ax
    ny_ref[...] = ay
    nz_ref[...] = az


@jax.jit
def _knn_pallas(cx, cy, cz, px, py, pz, pxt, pyt, pzt, pxyz):
    outs = pl.pallas_call(
        _knn_kernel,
        grid=(_B, _C // _CB),
        in_specs=(
            [pl.BlockSpec((1, 1, _CB), lambda b, c: (b * (_C // _CB) + c, 0, 0),
                          memory_space=pltpu.SMEM)] * 3
            + [pl.BlockSpec((1, _R, 128), lambda b, c: (b, 0, 0))] * 3
            + [pl.BlockSpec((1, 128, _R), lambda b, c: (b, 0, 0))] * 3
            + [pl.BlockSpec((1, _R, 3 * 128), lambda b, c: (b, 0, 0))]
        ),
        out_specs=[pl.BlockSpec((_CB, _KNN),
                                lambda b, c: (b * (_C // _CB) + c, 0))] * 3,
        out_shape=[jax.ShapeDtypeStruct((_B * _C, _KNN), jnp.float32)] * 3,
        scratch_shapes=[pltpu.VMEM((_CB, _R, 128), jnp.float32)],
        compiler_params=pltpu.CompilerParams(
            dimension_semantics=("parallel", "parallel")),
    )(cx, cy, cz, px, py, pz, pxt, pyt, pzt, pxyz)
    return outs


def _gelu(x):
    return jax.nn.gelu(x, approximate=True)


def _ln(x, g, b):
    m = jnp.mean(x, axis=-1, keepdims=True)
    v = jnp.mean((x - m) ** 2, axis=-1, keepdims=True)
    return (x - m) / jnp.sqrt(v + 1e-5) * g + b


def _knn_jnp(centers, points, k):
    cn = jnp.sum(centers ** 2, axis=-1)[:, :, None]
    pn = jnp.sum(points ** 2, axis=-1)[:, None, :]
    dot = jnp.einsum('bkd,bnd->bkn', centers, points,
                     precision=jax.lax.Precision.HIGHEST)
    d = cn + pn - 2.0 * dot
    _, idx = jax.lax.top_k(-d, k)
    knn_pts = jax.vmap(lambda p, i: jnp.take(p, i, axis=0))(points, idx)
    return knn_pts


def kernel(point_cloud, W1, b1, g1, be1, W2, b2, g2, be2, W3, b3, g3, be3,
           W4, b4, Wc1, bc1, Wc2, bc2):
    px = point_cloud[..., 0].reshape(_B, _R, 128)
    py = point_cloud[..., 1].reshape(_B, _R, 128)
    pz = point_cloud[..., 2].reshape(_B, _R, 128)
    cx, cy, cz = _fps(px, py, pz)
    centers = jnp.concatenate([cx, cy, cz], axis=1).reshape(_B, _C, 3)

    pxt = jnp.swapaxes(px, 1, 2)
    pyt = jnp.swapaxes(py, 1, 2)
    pzt = jnp.swapaxes(pz, 1, 2)
    pxyz = jnp.concatenate([px, py, pz], axis=2)
    nx, ny, nz = _knn_pallas(cx.reshape(-1, 1, _CB), cy.reshape(-1, 1, _CB),
                             cz.reshape(-1, 1, _CB), px, py, pz,
                             pxt, pyt, pzt, pxyz)
    normed = jnp.stack([nx, ny, nz], axis=-1).reshape(_B, _C, _KNN, 3)
    center_emb = _gelu(centers @ Wc1 + bc1) @ Wc2 + bc2
    h = _gelu(_ln(normed @ W1 + b1, g1, be1))
    h = _gelu(_ln(h @ W2 + b2, g2, be2))
    h = _gelu(_ln(h @ W3 + b3, g3, be3))
    h = jnp.max(h, axis=-2)
    knn_emb = h @ W4 + b4
    return (center_emb + knn_emb, centers, normed)


# speedup vs baseline: 12.8661x; 1.2792x over previous
import functools

import jax
import jax.numpy as jnp
from jax.experimental import pallas as pl
from jax.experimental.pallas import tpu as pltpu

_B, _N, _C, _KNN, _D = 4, 16384, 512, 32, 768
_R = _N // 128
_CB = 64


def _fps_kernel(px_ref, py_ref, pz_ref, cx_ref, cy_ref, cz_ref):
    flat = (jax.lax.broadcasted_iota(jnp.int32, (_R, 128), 0) * 128
            + jax.lax.broadcasted_iota(jnp.int32, (_R, 128), 1))
    px = px_ref[0]
    py = py_ref[0]
    pz = pz_ref[0]
    lx0 = px[0, 0]
    ly0 = py[0, 0]
    lz0 = pz[0, 0]
    cx_ref[pl.ds(0, 1), :] = lx0[None, None]
    cy_ref[pl.ds(0, 1), :] = ly0[None, None]
    cz_ref[pl.ds(0, 1), :] = lz0[None, None]
    dists0 = jnp.full((_R, 128), jnp.inf, jnp.float32)

    def body(i, carry):
        dists, lx, ly, lz = carry
        dx = px - lx
        dy = py - ly
        dz = pz - lz
        d = (dx * dx + dy * dy) + dz * dz
        dists = jnp.minimum(dists, d)
        m = jnp.max(dists)
        sel = jnp.where(dists == m, flat, jnp.int32(1 << 30))
        idx = jnp.min(sel)
        msk = flat == idx
        nlx = jnp.sum(jnp.where(msk, px, 0.0))
        nly = jnp.sum(jnp.where(msk, py, 0.0))
        nlz = jnp.sum(jnp.where(msk, pz, 0.0))
        cx_ref[pl.ds(i, 1), :] = nlx[None, None]
        cy_ref[pl.ds(i, 1), :] = nly[None, None]
        cz_ref[pl.ds(i, 1), :] = nlz[None, None]
        return dists, nlx, nly, nlz

    jax.lax.fori_loop(1, _C, body, (dists0, lx0, ly0, lz0))


@jax.jit
def _fps(px, py, pz):
    cs = pl.pallas_call(
        _fps_kernel,
        grid=(_B,),
        in_specs=[pl.BlockSpec((1, _R, 128), lambda b: (b, 0, 0))] * 3,
        out_specs=[pl.BlockSpec((_C, 1), lambda b: (b, 0))] * 3,
        out_shape=[jax.ShapeDtypeStruct((_B * _C, 1), jnp.float32)] * 3,
        compiler_params=pltpu.CompilerParams(
            dimension_semantics=("parallel",)),
    )(px, py, pz)
    return cs


def _knn_kernel(cx_ref, cy_ref, cz_ref, px_ref, py_ref, pz_ref,
                pxt_ref, pyt_ref, pzt_ref, pxyz_ref,
                nx_ref, ny_ref, nz_ref, d_scr):
    px = px_ref[0]
    py = py_ref[0]
    pz = pz_ref[0]
    pn = (px * px + py * py) + pz * pz
    pxt = pxt_ref[0]
    pyt = pyt_ref[0]
    pzt = pzt_ref[0]
    pnt = (pxt * pxt + pyt * pyt) + pzt * pzt
    pxb = px.astype(jnp.bfloat16).astype(jnp.float32)
    pyb = py.astype(jnp.bfloat16).astype(jnp.float32)
    pzb = pz.astype(jnp.bfloat16).astype(jnp.float32)
    pxtb = pxt.astype(jnp.bfloat16).astype(jnp.float32)
    pytb = pyt.astype(jnp.bfloat16).astype(jnp.float32)
    pztb = pzt.astype(jnp.bfloat16).astype(jnp.float32)
    lane = jax.lax.broadcasted_iota(jnp.int32, (1, 128), 1)
    lane3 = jax.lax.broadcasted_iota(jnp.int32, (1, 3 * 128), 1)
    lane32 = jax.lax.broadcasted_iota(jnp.int32, (1, _KNN), 1)
    BIGI = jnp.int32(1 << 30)
    INF = jnp.float32(jnp.inf)
    cxs = [cx_ref[0, 0, i] for i in range(_CB)]
    cys = [cy_ref[0, 0, i] for i in range(_CB)]
    czs = [cz_ref[0, 0, i] for i in range(_CB)]
    rows_R = []
    for i in range(_CB):
        cn = (cxs[i] * cxs[i] + cys[i] * cys[i]) + czs[i] * czs[i]
        cxb = cxs[i].astype(jnp.bfloat16).astype(jnp.float32)
        cyb = cys[i].astype(jnp.bfloat16).astype(jnp.float32)
        czb = czs[i].astype(jnp.bfloat16).astype(jnp.float32)
        dot = (cxb * pxb + cyb * pyb) + czb * pzb
        d_scr[i] = cn + pn - 2.0 * dot
        dott = (cxb * pxtb + cyb * pytb) + czb * pztb
        dt = cn + pnt - 2.0 * dott
        rows_R.append(jnp.min(dt, axis=0, keepdims=True))
    R8 = jnp.concatenate(rows_R, axis=0)
    cx8 = jnp.concatenate([c[None, None] for c in cxs], axis=0)
    cy8 = jnp.concatenate([c[None, None] for c in cys], axis=0)
    cz8 = jnp.concatenate([c[None, None] for c in czs], axis=0)

    def pass_body(k, carry):
        R8, ax, ay, az = carry
        m8 = jnp.min(R8, axis=1, keepdims=True)
        g8 = jnp.min(jnp.where(R8 == m8, lane, BIGI), axis=1,
                     keepdims=True)
        gs = [g8[i, 0] for i in range(_CB)]
        rows = jnp.concatenate(
            [d_scr[i, pl.ds(gs[i], 1), :] for i in range(_CB)], axis=0)
        c8 = jnp.min(jnp.where(rows == m8, lane, BIGI), axis=1,
                     keepdims=True)
        prows = jnp.concatenate(
            [pxyz_ref[0, pl.ds(gs[i], 1), :] for i in range(_CB)], axis=0)
        nx8 = jnp.sum(jnp.where(lane3 == c8, prows, 0.0), axis=1,
                      keepdims=True)
        ny8 = jnp.sum(jnp.where(lane3 == c8 + 128, prows, 0.0), axis=1,
                      keepdims=True)
        nz8 = jnp.sum(jnp.where(lane3 == c8 + 256, prows, 0.0), axis=1,
                      keepdims=True)
        km = lane32 == k
        ax = jnp.where(km, nx8 - cx8, ax)
        ay = jnp.where(km, ny8 - cy8, ay)
        az = jnp.where(km, nz8 - cz8, az)
        rows2 = jnp.where(lane == c8, INF, rows)
        for i in range(_CB):
            d_scr[i, pl.ds(gs[i], 1), :] = rows2[i:i + 1, :]
        R8 = jnp.where(lane == g8, jnp.min(rows2, axis=1, keepdims=True), R8)
        return R8, ax, ay, az

    z32 = jnp.zeros((_CB, _KNN), jnp.float32)
    _, ax, ay, az = jax.lax.fori_loop(0, _KNN, pass_body, (R8, z32, z32, z32))
    nx_ref[...] = ax
    ny_ref[...] = ay
    nz_ref[...] = az


@jax.jit
def _knn_pallas(cx, cy, cz, px, py, pz, pxt, pyt, pzt, pxyz):
    outs = pl.pallas_call(
        _knn_kernel,
        grid=(_B, _C // _CB),
        in_specs=(
            [pl.BlockSpec((1, 1, _CB), lambda b, c: (b * (_C // _CB) + c, 0, 0),
                          memory_space=pltpu.SMEM)] * 3
            + [pl.BlockSpec((1, _R, 128), lambda b, c: (b, 0, 0))] * 3
            + [pl.BlockSpec((1, 128, _R), lambda b, c: (b, 0, 0))] * 3
            + [pl.BlockSpec((1, _R, 3 * 128), lambda b, c: (b, 0, 0))]
        ),
        out_specs=[pl.BlockSpec((_CB, _KNN),
                                lambda b, c: (b * (_C // _CB) + c, 0))] * 3,
        out_shape=[jax.ShapeDtypeStruct((_B * _C, _KNN), jnp.float32)] * 3,
        scratch_shapes=[pltpu.VMEM((_CB, _R, 128), jnp.float32)],
        compiler_params=pltpu.CompilerParams(
            dimension_semantics=("parallel", "parallel")),
    )(cx, cy, cz, px, py, pz, pxt, pyt, pzt, pxyz)
    return outs


def _gelu(x):
    return jax.nn.gelu(x, approximate=True)


def _ln(x, g, b):
    m = jnp.mean(x, axis=-1, keepdims=True)
    v = jnp.mean((x - m) ** 2, axis=-1, keepdims=True)
    return (x - m) / jnp.sqrt(v + 1e-5) * g + b


def _knn_jnp(centers, points, k):
    cn = jnp.sum(centers ** 2, axis=-1)[:, :, None]
    pn = jnp.sum(points ** 2, axis=-1)[:, None, :]
    dot = jnp.einsum('bkd,bnd->bkn', centers, points,
                     precision=jax.lax.Precision.HIGHEST)
    d = cn + pn - 2.0 * dot
    _, idx = jax.lax.top_k(-d, k)
    knn_pts = jax.vmap(lambda p, i: jnp.take(p, i, axis=0))(points, idx)
    return knn_pts


def kernel(point_cloud, W1, b1, g1, be1, W2, b2, g2, be2, W3, b3, g3, be3,
           W4, b4, Wc1, bc1, Wc2, bc2):
    px = point_cloud[..., 0].reshape(_B, _R, 128)
    py = point_cloud[..., 1].reshape(_B, _R, 128)
    pz = point_cloud[..., 2].reshape(_B, _R, 128)
    cx, cy, cz = _fps(px, py, pz)
    centers = jnp.concatenate([cx, cy, cz], axis=1).reshape(_B, _C, 3)

    pxt = jnp.swapaxes(px, 1, 2)
    pyt = jnp.swapaxes(py, 1, 2)
    pzt = jnp.swapaxes(pz, 1, 2)
    pxyz = jnp.concatenate([px, py, pz], axis=2)
    nx, ny, nz = _knn_pallas(cx.reshape(-1, 1, _CB), cy.reshape(-1, 1, _CB),
                             cz.reshape(-1, 1, _CB), px, py, pz,
                             pxt, pyt, pzt, pxyz)
    normed = jnp.stack([nx, ny, nz], axis=-1).reshape(_B, _C, _KNN, 3)
    center_emb = _gelu(centers @ Wc1 + bc1) @ Wc2 + bc2
    h = _gelu(_ln(normed @ W1 + b1, g1, be1))
    h = _gelu(_ln(h @ W2 + b2, g2, be2))
    h = _gelu(_ln(h @ W3 + b3, g3, be3))
    h = jnp.max(h, axis=-2)
    knn_emb = h @ W4 + b4
    return (center_emb + knn_emb, centers, normed)


# speedup vs baseline: 14.6660x; 1.1399x over previous
import functools

import jax
import jax.numpy as jnp
from jax.experimental import pallas as pl
from jax.experimental.pallas import tpu as pltpu

_B, _N, _C, _KNN, _D = 4, 16384, 512, 32, 768
_R = _N // 128
_CB = 128


def _fps_kernel(px_ref, py_ref, pz_ref, cx_ref, cy_ref, cz_ref):
    flat = (jax.lax.broadcasted_iota(jnp.int32, (_R, 128), 0) * 128
            + jax.lax.broadcasted_iota(jnp.int32, (_R, 128), 1))
    px = px_ref[0]
    py = py_ref[0]
    pz = pz_ref[0]
    lx0 = px[0, 0]
    ly0 = py[0, 0]
    lz0 = pz[0, 0]
    cx_ref[pl.ds(0, 1), :] = lx0[None, None]
    cy_ref[pl.ds(0, 1), :] = ly0[None, None]
    cz_ref[pl.ds(0, 1), :] = lz0[None, None]
    dists0 = jnp.full((_R, 128), jnp.inf, jnp.float32)

    def body(i, carry):
        dists, lx, ly, lz = carry
        dx = px - lx
        dy = py - ly
        dz = pz - lz
        d = (dx * dx + dy * dy) + dz * dz
        dists = jnp.minimum(dists, d)
        m = jnp.max(dists)
        sel = jnp.where(dists == m, flat, jnp.int32(1 << 30))
        idx = jnp.min(sel)
        msk = flat == idx
        nlx = jnp.sum(jnp.where(msk, px, 0.0))
        nly = jnp.sum(jnp.where(msk, py, 0.0))
        nlz = jnp.sum(jnp.where(msk, pz, 0.0))
        cx_ref[pl.ds(i, 1), :] = nlx[None, None]
        cy_ref[pl.ds(i, 1), :] = nly[None, None]
        cz_ref[pl.ds(i, 1), :] = nlz[None, None]
        return dists, nlx, nly, nlz

    jax.lax.fori_loop(1, _C, body, (dists0, lx0, ly0, lz0))


@jax.jit
def _fps(px, py, pz):
    cs = pl.pallas_call(
        _fps_kernel,
        grid=(_B,),
        in_specs=[pl.BlockSpec((1, _R, 128), lambda b: (b, 0, 0))] * 3,
        out_specs=[pl.BlockSpec((_C, 1), lambda b: (b, 0))] * 3,
        out_shape=[jax.ShapeDtypeStruct((_B * _C, 1), jnp.float32)] * 3,
        compiler_params=pltpu.CompilerParams(
            dimension_semantics=("parallel",)),
    )(px, py, pz)
    return cs


def _knn_kernel(cx_ref, cy_ref, cz_ref, px_ref, py_ref, pz_ref,
                pxt_ref, pyt_ref, pzt_ref, pxyz_ref,
                nx_ref, ny_ref, nz_ref, d_scr):
    px = px_ref[0]
    py = py_ref[0]
    pz = pz_ref[0]
    pn = (px * px + py * py) + pz * pz
    pxt = pxt_ref[0]
    pyt = pyt_ref[0]
    pzt = pzt_ref[0]
    pnt = (pxt * pxt + pyt * pyt) + pzt * pzt
    pxb = px.astype(jnp.bfloat16).astype(jnp.float32)
    pyb = py.astype(jnp.bfloat16).astype(jnp.float32)
    pzb = pz.astype(jnp.bfloat16).astype(jnp.float32)
    pxtb = pxt.astype(jnp.bfloat16).astype(jnp.float32)
    pytb = pyt.astype(jnp.bfloat16).astype(jnp.float32)
    pztb = pzt.astype(jnp.bfloat16).astype(jnp.float32)
    lane = jax.lax.broadcasted_iota(jnp.int32, (1, 128), 1)
    lane3 = jax.lax.broadcasted_iota(jnp.int32, (1, 3 * 128), 1)
    lane32 = jax.lax.broadcasted_iota(jnp.int32, (1, _KNN), 1)
    BIGI = jnp.int32(1 << 30)
    INF = jnp.float32(jnp.inf)
    cxs = [cx_ref[0, 0, i] for i in range(_CB)]
    cys = [cy_ref[0, 0, i] for i in range(_CB)]
    czs = [cz_ref[0, 0, i] for i in range(_CB)]
    rows_R = []
    for i in range(_CB):
        cn = (cxs[i] * cxs[i] + cys[i] * cys[i]) + czs[i] * czs[i]
        cxb = cxs[i].astype(jnp.bfloat16).astype(jnp.float32)
        cyb = cys[i].astype(jnp.bfloat16).astype(jnp.float32)
        czb = czs[i].astype(jnp.bfloat16).astype(jnp.float32)
        dot = (cxb * pxb + cyb * pyb) + czb * pzb
        d_scr[i] = cn + pn - 2.0 * dot
        dott = (cxb * pxtb + cyb * pytb) + czb * pztb
        dt = cn + pnt - 2.0 * dott
        rows_R.append(jnp.min(dt, axis=0, keepdims=True))
    R8 = jnp.concatenate(rows_R, axis=0)
    cx8 = jnp.concatenate([c[None, None] for c in cxs], axis=0)
    cy8 = jnp.concatenate([c[None, None] for c in cys], axis=0)
    cz8 = jnp.concatenate([c[None, None] for c in czs], axis=0)

    def pass_body(k, carry):
        R8, ax, ay, az = carry
        m8 = jnp.min(R8, axis=1, keepdims=True)
        g8 = jnp.min(jnp.where(R8 == m8, lane, BIGI), axis=1,
                     keepdims=True)
        gs = [g8[i, 0] for i in range(_CB)]
        rows = jnp.concatenate(
            [d_scr[i, pl.ds(gs[i], 1), :] for i in range(_CB)], axis=0)
        c8 = jnp.min(jnp.where(rows == m8, lane, BIGI), axis=1,
                     keepdims=True)
        prows = jnp.concatenate(
            [pxyz_ref[0, pl.ds(gs[i], 1), :] for i in range(_CB)], axis=0)
        nx8 = jnp.sum(jnp.where(lane3 == c8, prows, 0.0), axis=1,
                      keepdims=True)
        ny8 = jnp.sum(jnp.where(lane3 == c8 + 128, prows, 0.0), axis=1,
                      keepdims=True)
        nz8 = jnp.sum(jnp.where(lane3 == c8 + 256, prows, 0.0), axis=1,
                      keepdims=True)
        km = lane32 == k
        ax = jnp.where(km, nx8 - cx8, ax)
        ay = jnp.where(km, ny8 - cy8, ay)
        az = jnp.where(km, nz8 - cz8, az)
        rows2 = jnp.where(lane == c8, INF, rows)
        for i in range(_CB):
            d_scr[i, pl.ds(gs[i], 1), :] = rows2[i:i + 1, :]
        R8 = jnp.where(lane == g8, jnp.min(rows2, axis=1, keepdims=True), R8)
        return R8, ax, ay, az

    z32 = jnp.zeros((_CB, _KNN), jnp.float32)
    _, ax, ay, az = jax.lax.fori_loop(0, _KNN, pass_body, (R8, z32, z32, z32))
    nx_ref[...] = ax
    ny_ref[...] = ay
    nz_ref[...] = az


@jax.jit
def _knn_pallas(cx, cy, cz, px, py, pz, pxt, pyt, pzt, pxyz):
    outs = pl.pallas_call(
        _knn_kernel,
        grid=(_B, _C // _CB),
        in_specs=(
            [pl.BlockSpec((1, 1, _CB), lambda b, c: (b * (_C // _CB) + c, 0, 0),
                          memory_space=pltpu.SMEM)] * 3
            + [pl.BlockSpec((1, _R, 128), lambda b, c: (b, 0, 0))] * 3
            + [pl.BlockSpec((1, 128, _R), lambda b, c: (b, 0, 0))] * 3
            + [pl.BlockSpec((1, _R, 3 * 128), lambda b, c: (b, 0, 0))]
        ),
        out_specs=[pl.BlockSpec((_CB, _KNN),
                                lambda b, c: (b * (_C // _CB) + c, 0))] * 3,
        out_shape=[jax.ShapeDtypeStruct((_B * _C, _KNN), jnp.float32)] * 3,
        scratch_shapes=[pltpu.VMEM((_CB, _R, 128), jnp.float32)],
        compiler_params=pltpu.CompilerParams(
            dimension_semantics=("parallel", "parallel")),
    )(cx, cy, cz, px, py, pz, pxt, pyt, pzt, pxyz)
    return outs


def _gelu(x):
    return jax.nn.gelu(x, approximate=True)


def _ln(x, g, b):
    m = jnp.mean(x, axis=-1, keepdims=True)
    v = jnp.mean((x - m) ** 2, axis=-1, keepdims=True)
    return (x - m) / jnp.sqrt(v + 1e-5) * g + b


def _knn_jnp(centers, points, k):
    cn = jnp.sum(centers ** 2, axis=-1)[:, :, None]
    pn = jnp.sum(points ** 2, axis=-1)[:, None, :]
    dot = jnp.einsum('bkd,bnd->bkn', centers, points,
                     precision=jax.lax.Precision.HIGHEST)
    d = cn + pn - 2.0 * dot
    _, idx = jax.lax.top_k(-d, k)
    knn_pts = jax.vmap(lambda p, i: jnp.take(p, i, axis=0))(points, idx)
    return knn_pts


def kernel(point_cloud, W1, b1, g1, be1, W2, b2, g2, be2, W3, b3, g3, be3,
           W4, b4, Wc1, bc1, Wc2, bc2):
    px = point_cloud[..., 0].reshape(_B, _R, 128)
    py = point_cloud[..., 1].reshape(_B, _R, 128)
    pz = point_cloud[..., 2].reshape(_B, _R, 128)
    cx, cy, cz = _fps(px, py, pz)
    centers = jnp.concatenate([cx, cy, cz], axis=1).reshape(_B, _C, 3)

    pxt = jnp.swapaxes(px, 1, 2)
    pyt = jnp.swapaxes(py, 1, 2)
    pzt = jnp.swapaxes(pz, 1, 2)
    pxyz = jnp.concatenate([px, py, pz], axis=2)
    nx, ny, nz = _knn_pallas(cx.reshape(-1, 1, _CB), cy.reshape(-1, 1, _CB),
                             cz.reshape(-1, 1, _CB), px, py, pz,
                             pxt, pyt, pzt, pxyz)
    normed = jnp.stack([nx, ny, nz], axis=-1).reshape(_B, _C, _KNN, 3)
    center_emb = _gelu(centers @ Wc1 + bc1) @ Wc2 + bc2
    h = _gelu(_ln(normed @ W1 + b1, g1, be1))
    h = _gelu(_ln(h @ W2 + b2, g2, be2))
    h = _gelu(_ln(h @ W3 + b3, g3, be3))
    h = jnp.max(h, axis=-2)
    knn_emb = h @ W4 + b4
    return (center_emb + knn_emb, centers, normed)


# speedup vs baseline: 16.9313x; 1.1545x over previous
import functools

import jax
import jax.numpy as jnp
from jax.experimental import pallas as pl
from jax.experimental.pallas import tpu as pltpu

_B, _N, _C, _KNN, _D = 4, 16384, 512, 32, 768
_R = _N // 128
_CB = 128


def _fps_kernel(px_ref, py_ref, pz_ref, pxyz_ref, cx_ref, cy_ref, cz_ref):
    flat = (jax.lax.broadcasted_iota(jnp.int32, (_R, 128), 0) * 128
            + jax.lax.broadcasted_iota(jnp.int32, (_R, 128), 1))
    lane3 = jax.lax.broadcasted_iota(jnp.int32, (1, 3 * 128), 1)
    BIGI = jnp.int32(1 << 30)
    init = []
    for b in range(_B):
        px = px_ref[b]
        py = py_ref[b]
        pz = pz_ref[b]
        lx0 = px[0, 0]
        ly0 = py[0, 0]
        lz0 = pz[0, 0]
        cx_ref[pl.ds(b * _C, 1), :] = lx0[None, None]
        cy_ref[pl.ds(b * _C, 1), :] = ly0[None, None]
        cz_ref[pl.ds(b * _C, 1), :] = lz0[None, None]
        init += [jnp.full((_R, 128), jnp.inf, jnp.float32), lx0, ly0, lz0]

    def body(i, carry):
        out = []
        for b in range(_B):
            dists, lx, ly, lz = carry[4 * b:4 * b + 4]
            dx = px_ref[b] - lx
            dy = py_ref[b] - ly
            dz = pz_ref[b] - lz
            d = (dx * dx + dy * dy) + dz * dz
            dists = jnp.minimum(dists, d)
            m = jnp.max(dists)
            idx = jnp.min(jnp.where(dists == m, flat, BIGI))
            r = idx // 128
            c = idx - r * 128
            prow = pxyz_ref[b, pl.ds(r, 1), :]
            nlx = jnp.sum(jnp.where(lane3 == c, prow, 0.0))
            nly = jnp.sum(jnp.where(lane3 == c + 128, prow, 0.0))
            nlz = jnp.sum(jnp.where(lane3 == c + 256, prow, 0.0))
            cx_ref[pl.ds(b * _C + i, 1), :] = nlx[None, None]
            cy_ref[pl.ds(b * _C + i, 1), :] = nly[None, None]
            cz_ref[pl.ds(b * _C + i, 1), :] = nlz[None, None]
            out += [dists, nlx, nly, nlz]
        return tuple(out)

    jax.lax.fori_loop(1, _C, body, tuple(init))


@jax.jit
def _fps(px, py, pz, pxyz):
    cs = pl.pallas_call(
        _fps_kernel,
        out_specs=[pl.BlockSpec((_B * _C, 1), lambda: (0, 0))] * 3,
        out_shape=[jax.ShapeDtypeStruct((_B * _C, 1), jnp.float32)] * 3,
    )(px, py, pz, pxyz)
    return cs


def _knn_kernel(cx_ref, cy_ref, cz_ref, px_ref, py_ref, pz_ref,
                pxt_ref, pyt_ref, pzt_ref, pxyz_ref,
                nx_ref, ny_ref, nz_ref, d_scr):
    px = px_ref[0]
    py = py_ref[0]
    pz = pz_ref[0]
    pn = (px * px + py * py) + pz * pz
    pxt = pxt_ref[0]
    pyt = pyt_ref[0]
    pzt = pzt_ref[0]
    pnt = (pxt * pxt + pyt * pyt) + pzt * pzt
    pxb = px.astype(jnp.bfloat16).astype(jnp.float32)
    pyb = py.astype(jnp.bfloat16).astype(jnp.float32)
    pzb = pz.astype(jnp.bfloat16).astype(jnp.float32)
    pxtb = pxt.astype(jnp.bfloat16).astype(jnp.float32)
    pytb = pyt.astype(jnp.bfloat16).astype(jnp.float32)
    pztb = pzt.astype(jnp.bfloat16).astype(jnp.float32)
    lane = jax.lax.broadcasted_iota(jnp.int32, (1, 128), 1)
    lane3 = jax.lax.broadcasted_iota(jnp.int32, (1, 3 * 128), 1)
    lane32 = jax.lax.broadcasted_iota(jnp.int32, (1, _KNN), 1)
    BIGI = jnp.int32(1 << 30)
    INF = jnp.float32(jnp.inf)
    cxs = [cx_ref[0, 0, i] for i in range(_CB)]
    cys = [cy_ref[0, 0, i] for i in range(_CB)]
    czs = [cz_ref[0, 0, i] for i in range(_CB)]
    rows_R = []
    for i in range(_CB):
        cn = (cxs[i] * cxs[i] + cys[i] * cys[i]) + czs[i] * czs[i]
        cxb = cxs[i].astype(jnp.bfloat16).astype(jnp.float32)
        cyb = cys[i].astype(jnp.bfloat16).astype(jnp.float32)
        czb = czs[i].astype(jnp.bfloat16).astype(jnp.float32)
        dot = (cxb * pxb + cyb * pyb) + czb * pzb
        d_scr[i] = cn + pn - 2.0 * dot
        dott = (cxb * pxtb + cyb * pytb) + czb * pztb
        dt = cn + pnt - 2.0 * dott
        rows_R.append(jnp.min(dt, axis=0, keepdims=True))
    R8 = jnp.concatenate(rows_R, axis=0)
    cx8 = jnp.concatenate([c[None, None] for c in cxs], axis=0)
    cy8 = jnp.concatenate([c[None, None] for c in cys], axis=0)
    cz8 = jnp.concatenate([c[None, None] for c in czs], axis=0)

    def pass_body(k, carry):
        R8, ax, ay, az = carry
        m8 = jnp.min(R8, axis=1, keepdims=True)
        g8 = jnp.min(jnp.where(R8 == m8, lane, BIGI), axis=1,
                     keepdims=True)
        gs = [g8[i, 0] for i in range(_CB)]
        rows = jnp.concatenate(
            [d_scr[i, pl.ds(gs[i], 1), :] for i in range(_CB)], axis=0)
        c8 = jnp.min(jnp.where(rows == m8, lane, BIGI), axis=1,
                     keepdims=True)
        prows = jnp.concatenate(
            [pxyz_ref[0, pl.ds(gs[i], 1), :] for i in range(_CB)], axis=0)
        nx8 = jnp.sum(jnp.where(lane3 == c8, prows, 0.0), axis=1,
                      keepdims=True)
        ny8 = jnp.sum(jnp.where(lane3 == c8 + 128, prows, 0.0), axis=1,
                      keepdims=True)
        nz8 = jnp.sum(jnp.where(lane3 == c8 + 256, prows, 0.0), axis=1,
                      keepdims=True)
        km = lane32 == k
        ax = jnp.where(km, nx8 - cx8, ax)
        ay = jnp.where(km, ny8 - cy8, ay)
        az = jnp.where(km, nz8 - cz8, az)
        rows2 = jnp.where(lane == c8, INF, rows)
        for i in range(_CB):
            d_scr[i, pl.ds(gs[i], 1), :] = rows2[i:i + 1, :]
        R8 = jnp.where(lane == g8, jnp.min(rows2, axis=1, keepdims=True), R8)
        return R8, ax, ay, az

    z32 = jnp.zeros((_CB, _KNN), jnp.float32)
    _, ax, ay, az = jax.lax.fori_loop(0, _KNN, pass_body, (R8, z32, z32, z32))
    nx_ref[...] = ax
    ny_ref[...] = ay
    nz_ref[...] = az


@jax.jit
def _knn_pallas(cx, cy, cz, px, py, pz, pxt, pyt, pzt, pxyz):
    outs = pl.pallas_call(
        _knn_kernel,
        grid=(_B, _C // _CB),
        in_specs=(
            [pl.BlockSpec((1, 1, _CB), lambda b, c: (b * (_C // _CB) + c, 0, 0),
                          memory_space=pltpu.SMEM)] * 3
            + [pl.BlockSpec((1, _R, 128), lambda b, c: (b, 0, 0))] * 3
            + [pl.BlockSpec((1, 128, _R), lambda b, c: (b, 0, 0))] * 3
            + [pl.BlockSpec((1, _R, 3 * 128), lambda b, c: (b, 0, 0))]
        ),
        out_specs=[pl.BlockSpec((_CB, _KNN),
                                lambda b, c: (b * (_C // _CB) + c, 0))] * 3,
        out_shape=[jax.ShapeDtypeStruct((_B * _C, _KNN), jnp.float32)] * 3,
        scratch_shapes=[pltpu.VMEM((_CB, _R, 128), jnp.float32)],
        compiler_params=pltpu.CompilerParams(
            dimension_semantics=("parallel", "parallel")),
    )(cx, cy, cz, px, py, pz, pxt, pyt, pzt, pxyz)
    return outs


def _gelu(x):
    return jax.nn.gelu(x, approximate=True)


def _ln(x, g, b):
    m = jnp.mean(x, axis=-1, keepdims=True)
    v = jnp.mean((x - m) ** 2, axis=-1, keepdims=True)
    return (x - m) / jnp.sqrt(v + 1e-5) * g + b


def _knn_jnp(centers, points, k):
    cn = jnp.sum(centers ** 2, axis=-1)[:, :, None]
    pn = jnp.sum(points ** 2, axis=-1)[:, None, :]
    dot = jnp.einsum('bkd,bnd->bkn', centers, points,
                     precision=jax.lax.Precision.HIGHEST)
    d = cn + pn - 2.0 * dot
    _, idx = jax.lax.top_k(-d, k)
    knn_pts = jax.vmap(lambda p, i: jnp.take(p, i, axis=0))(points, idx)
    return knn_pts


def kernel(point_cloud, W1, b1, g1, be1, W2, b2, g2, be2, W3, b3, g3, be3,
           W4, b4, Wc1, bc1, Wc2, bc2):
    px = point_cloud[..., 0].reshape(_B, _R, 128)
    py = point_cloud[..., 1].reshape(_B, _R, 128)
    pz = point_cloud[..., 2].reshape(_B, _R, 128)
    pxt = jnp.swapaxes(px, 1, 2)
    pyt = jnp.swapaxes(py, 1, 2)
    pzt = jnp.swapaxes(pz, 1, 2)
    pxyz = jnp.concatenate([px, py, pz], axis=2)
    cx, cy, cz = _fps(px, py, pz, pxyz)
    centers = jnp.concatenate([cx, cy, cz], axis=1).reshape(_B, _C, 3)
    nx, ny, nz = _knn_pallas(cx.reshape(-1, 1, _CB), cy.reshape(-1, 1, _CB),
                             cz.reshape(-1, 1, _CB), px, py, pz,
                             pxt, pyt, pzt, pxyz)
    normed = jnp.stack([nx, ny, nz], axis=-1).reshape(_B, _C, _KNN, 3)
    center_emb = _gelu(centers @ Wc1 + bc1) @ Wc2 + bc2
    h = _gelu(_ln(normed @ W1 + b1, g1, be1))
    h = _gelu(_ln(h @ W2 + b2, g2, be2))
    h = _gelu(_ln(h @ W3 + b3, g3, be3))
    h = jnp.max(h, axis=-2)
    knn_emb = h @ W4 + b4
    return (center_emb + knn_emb, centers, normed)
